# Initial kernel scaffold; baseline (speedup 1.0000x reference)
import functools
import jax
import jax.numpy as jnp
from jax import lax
from jax.experimental import pallas as pl
from jax.experimental.pallas import tpu as pltpu
from jax.experimental.pallas import tpu_sc as plsc

N = 10000
E = 320000
D = 128
H = 256
C = 5
CP = 16

NC = 2
NS = 16
NW = NC * NS
K = 80
RPT = N // NS


def _sc_agg_both(x, src3, dst3, zeros_init):
    nch = src3.shape[1]
    mesh = plsc.VectorSubcoreMesh(core_axis_name="c", subcore_axis_name="s")

    @functools.partial(
        pl.kernel,
        mesh=mesh,
        out_type=jax.ShapeDtypeStruct((NC, N, D), jnp.float32),
        scratch_types=[
            pltpu.VMEM((nch, K), jnp.int32),
            pltpu.VMEM((nch, K), jnp.int32),
            pltpu.VMEM((K, D), jnp.float32),
            pltpu.VMEM_SHARED((N, D), jnp.float32),
        ],
    )
    def k(x_hbm, src_hbm, dst_hbm, init_hbm, out_hbm, src_v, dst_v, rows_v, acc):
        cid = lax.axis_index("c")
        sid = lax.axis_index("s")
        wid = sid * NC + cid
        pltpu.sync_copy(init_hbm.at[pl.ds(sid * RPT, RPT)],
                        acc.at[pl.ds(sid * RPT, RPT)])
        pltpu.sync_copy(src_hbm.at[wid], src_v)
        pltpu.sync_copy(dst_hbm.at[wid], dst_v)
        plsc.subcore_barrier()

        def body(i, carry):
            pltpu.sync_copy(x_hbm.at[src_v.at[i]], rows_v)
            pltpu.sync_copy(rows_v, acc.at[dst_v.at[i]], add=True)
            return carry

        lax.fori_loop(0, nch, body, 0)
        plsc.subcore_barrier()
        pltpu.sync_copy(acc.at[pl.ds(sid * RPT, RPT)],
                        out_hbm.at[cid, pl.ds(sid * RPT, RPT)])

    return k(x, src3, dst3, zeros_init)


def _tc_mlp(partials, W1, b1, W2p):
    BN = 2000

    def body(p_ref, w1_ref, b1_ref, w2_ref, q_ref):
        a = p_ref[0] + p_ref[1]
        h = jnp.dot(a, w1_ref[...], preferred_element_type=jnp.float32)
        h = jnp.maximum(h + b1_ref[...], 0.0)
        q_ref[...] = jnp.dot(h, w2_ref[...], preferred_element_type=jnp.float32)

    return pl.pallas_call(
        body,
        grid=(N // BN,),
        in_specs=[
            pl.BlockSpec((NC, BN, D), lambda i: (0, i, 0)),
            pl.BlockSpec((D, H), lambda i: (0, 0)),
            pl.BlockSpec((1, H), lambda i: (0, 0)),
            pl.BlockSpec((H, CP), lambda i: (0, 0)),
        ],
        out_specs=pl.BlockSpec((BN, CP), lambda i: (i, 0)),
        out_shape=jax.ShapeDtypeStruct((N, CP), jnp.float32),
    )(partials, W1, b1, W2p)


def _sc_agg_one(q, src3, dst3, b2_init):
    nch = src3.shape[1]
    mesh = plsc.VectorSubcoreMesh(core_axis_name="c", subcore_axis_name="s")

    @functools.partial(
        pl.kernel,
        mesh=mesh,
        out_type=jax.ShapeDtypeStruct((N, CP), jnp.float32),
        scratch_types=[
            pltpu.VMEM((nch, K), jnp.int32),
            pltpu.VMEM((nch, K), jnp.int32),
            pltpu.VMEM((K, CP), jnp.float32),
            pltpu.VMEM_SHARED((N, CP), jnp.float32),
        ],
    )
    def k(q_hbm, src_hbm, dst_hbm, init_hbm, out_hbm, src_v, dst_v, rows_v, acc):
        cid = lax.axis_index("c")
        sid = lax.axis_index("s")

        @pl.when(cid == 0)
        def _():
            pltpu.sync_copy(init_hbm.at[pl.ds(sid * RPT, RPT)],
                            acc.at[pl.ds(sid * RPT, RPT)])
            pltpu.sync_copy(src_hbm.at[sid], src_v)
            pltpu.sync_copy(dst_hbm.at[sid], dst_v)
            plsc.subcore_barrier()

            def body(i, carry):
                pltpu.sync_copy(q_hbm.at[src_v.at[i]], rows_v)
                pltpu.sync_copy(rows_v, acc.at[dst_v.at[i]], add=True)
                return carry

            lax.fori_loop(0, nch, body, 0)
            plsc.subcore_barrier()
            pltpu.sync_copy(acc.at[pl.ds(sid * RPT, RPT)],
                            out_hbm.at[pl.ds(sid * RPT, RPT)])

    return k(q, src3, dst3, b2_init)


def kernel(x, edge_index, W1, b1, W2, b2):
    src = edge_index[0]
    dst = edge_index[1]
    nch1 = E // (NW * K)
    src1 = src.reshape(NW, nch1, K)
    dst1 = dst.reshape(NW, nch1, K)
    zeros_init = jnp.zeros((N, D), jnp.float32)
    partials = _sc_agg_both(x, src1, dst1, zeros_init)

    W2p = jnp.pad(W2, ((0, 0), (0, CP - C)))
    q = _tc_mlp(partials, W1, b1.reshape(1, H), W2p)

    nch2 = E // (NS * K)
    src2 = src.reshape(NS, nch2, K)
    dst2 = dst.reshape(NS, nch2, K)
    b2_init = jnp.broadcast_to(jnp.pad(b2, (0, CP - C)), (N, CP))
    out = _sc_agg_one(q, src2, dst2, b2_init)
    return out[:, :C]


# speedup vs baseline: 8.7661x; 8.7661x over previous
import functools
import jax
import jax.numpy as jnp
from jax import lax
from jax.experimental import pallas as pl
from jax.experimental.pallas import tpu as pltpu
from jax.experimental.pallas import tpu_sc as plsc

N = 10000
NP = 10240
E = 320000
D = 128
H = 256
C = 5
CP = 16

NC = 2
NS = 16
NW = NC * NS
K = 80
RPT = NP // NS


def _sc_agg_both(x, src3, dst3, zeros_init):
    nch = src3.shape[1]
    mesh = plsc.VectorSubcoreMesh(core_axis_name="c", subcore_axis_name="s")

    @functools.partial(
        pl.kernel,
        mesh=mesh,
        out_type=jax.ShapeDtypeStruct((NC, NP, D), jnp.float32),
        scratch_types=[
            pltpu.VMEM((nch, K), jnp.int32),
            pltpu.VMEM((nch, K), jnp.int32),
            pltpu.VMEM((K, D), jnp.float32),
            pltpu.VMEM_SHARED((NP, D), jnp.float32),
        ],
    )
    def k(x_hbm, src_hbm, dst_hbm, init_hbm, out_hbm, src_v, dst_v, rows_v, acc):
        cid = lax.axis_index("c")
        sid = lax.axis_index("s")
        wid = sid * NC + cid
        pltpu.sync_copy(init_hbm.at[pl.ds(sid * RPT, RPT)],
                        acc.at[pl.ds(sid * RPT, RPT)])
        pltpu.sync_copy(src_hbm.at[wid], src_v)
        pltpu.sync_copy(dst_hbm.at[wid], dst_v)
        plsc.subcore_barrier()

        def body(i, carry):
            pltpu.sync_copy(x_hbm.at[src_v.at[i]], rows_v)
            pltpu.sync_copy(rows_v, acc.at[dst_v.at[i]], add=True)
            return carry

        lax.fori_loop(0, nch, body, 0)
        plsc.subcore_barrier()
        pltpu.sync_copy(acc.at[pl.ds(sid * RPT, RPT)],
                        out_hbm.at[cid, pl.ds(sid * RPT, RPT)])

    return k(x, src3, dst3, zeros_init)


def _tc_mlp(partials, W1, b1, W2p):
    BN = 2048

    def body(p_ref, w1_ref, b1_ref, w2_ref, q_ref):
        a = p_ref[0] + p_ref[1]
        h = jnp.dot(a, w1_ref[...], preferred_element_type=jnp.float32)
        h = jnp.maximum(h + b1_ref[...], 0.0)
        q_ref[...] = jnp.dot(h, w2_ref[...], preferred_element_type=jnp.float32)

    return pl.pallas_call(
        body,
        grid=(NP // BN,),
        in_specs=[
            pl.BlockSpec((NC, BN, D), lambda i: (0, i, 0)),
            pl.BlockSpec((D, H), lambda i: (0, 0)),
            pl.BlockSpec((1, H), lambda i: (0, 0)),
            pl.BlockSpec((H, CP), lambda i: (0, 0)),
        ],
        out_specs=pl.BlockSpec((BN, CP), lambda i: (i, 0)),
        out_shape=jax.ShapeDtypeStruct((NP, CP), jnp.float32),
    )(partials, W1, b1, W2p)


def _sc_agg_one(q, src3, dst3, b2_init):
    nch = src3.shape[1]
    mesh = plsc.VectorSubcoreMesh(core_axis_name="c", subcore_axis_name="s")

    @functools.partial(
        pl.kernel,
        mesh=mesh,
        out_type=jax.ShapeDtypeStruct((NP, CP), jnp.float32),
        scratch_types=[
            pltpu.VMEM((nch, K), jnp.int32),
            pltpu.VMEM((nch, K), jnp.int32),
            pltpu.VMEM((K, CP), jnp.float32),
            pltpu.VMEM_SHARED((NP, CP), jnp.float32),
        ],
        compiler_params=pltpu.CompilerParams(use_tc_tiling_on_sc=False),
    )
    def k(q_hbm, src_hbm, dst_hbm, init_hbm, out_hbm, src_v, dst_v, rows_v, acc):
        cid = lax.axis_index("c")
        sid = lax.axis_index("s")

        @pl.when(cid == 0)
        def _():
            pltpu.sync_copy(init_hbm.at[pl.ds(sid * RPT, RPT)],
                            acc.at[pl.ds(sid * RPT, RPT)])
            pltpu.sync_copy(src_hbm.at[sid], src_v)
            pltpu.sync_copy(dst_hbm.at[sid], dst_v)
            plsc.subcore_barrier()

            def body(i, carry):
                pltpu.sync_copy(q_hbm.at[src_v.at[i]], rows_v)
                pltpu.sync_copy(rows_v, acc.at[dst_v.at[i]], add=True)
                return carry

            lax.fori_loop(0, nch, body, 0)
            plsc.subcore_barrier()
            pltpu.sync_copy(acc.at[pl.ds(sid * RPT, RPT)],
                            out_hbm.at[pl.ds(sid * RPT, RPT)])

    return k(q, src3, dst3, b2_init)


def kernel(x, edge_index, W1, b1, W2, b2):
    src = edge_index[0]
    dst = edge_index[1]
    nch1 = E // (NW * K)
    src1 = src.reshape(NW, nch1, K)
    dst1 = dst.reshape(NW, nch1, K)
    zeros_init = jnp.zeros((NP, D), jnp.float32)
    partials = _sc_agg_both(x, src1, dst1, zeros_init)

    W2p = jnp.pad(W2, ((0, 0), (0, CP - C)))
    q = _tc_mlp(partials, W1, b1.reshape(1, H), W2p)

    nch2 = E // (NS * K)
    src2 = src.reshape(NS, nch2, K)
    dst2 = dst.reshape(NS, nch2, K)
    b2_init = jnp.broadcast_to(jnp.pad(b2, (0, CP - C)), (NP, CP))
    out = _sc_agg_one(q, src2, dst2, b2_init)
    return out[:N, :C]
